# baseline (speedup 1.0000x reference)
import functools

import jax
import jax.numpy as jnp
from jax import lax
from jax.experimental import pallas as pl
from jax.experimental.pallas import tpu as pltpu
from jax.experimental.pallas import tpu_sc as plsc

_N = 16
_L = 16
_NS = 16
_ROWS = 4096
_D = 128
_CB = _D // _L
_RPW = _ROWS // _NS


_GATHER_DNUMS = lax.GatherDimensionNumbers(
    offset_dims=(), collapsed_slice_dims=(0,), start_index_map=(0,))


def _shuffle(v, idx):
    return lax.gather(v, idx[:, None], dimension_numbers=_GATHER_DNUMS,
                      slice_sizes=(1,),
                      mode=lax.GatherScatterMode.PROMISE_IN_BOUNDS)


def _allreduce_sum(v, lane):
    for sh in (8, 4, 2, 1):
        v = v + _shuffle(v, jnp.bitwise_xor(lane, sh))
    return v


def _sc_loss_kernel(im_hbm, s_hbm, bounds_hbm, sizes_hbm, out_hbm,
                    meta_v, sizes_v, chunk_v, acc_v, idx_v, srow_v, sall_v,
                    row_v, blk_v, shared_acc, shared_blk):
    sid = lax.axis_index("s")
    lane = lax.iota(jnp.int32, _L)

    pltpu.sync_copy(bounds_hbm, meta_v)
    pltpu.sync_copy(sizes_hbm, sizes_v)

    def _zero_row(r, _):
        for cb in range(_CB):
            acc_v[r, pl.ds(cb * _L, _L)] = jnp.zeros((_L,), jnp.float32)
        return 0
    lax.fori_loop(0, 2 * _N, _zero_row, 0)

    @pl.when(sid == 0)
    def _():
        pltpu.sync_copy(acc_v, shared_acc)
    plsc.subcore_barrier()

    base = sid * _RPW

    def _accumulate(src_hbm, srow, erow, acc_base):
        pltpu.sync_copy(src_hbm.at[pl.ds(base, _RPW)], chunk_v)
        starts_vec = meta_v[srow, :]
        ends_vec = meta_v[erow, :]
        for g in range(_N):
            lo = jnp.clip(starts_vec[g] - base, 0, _RPW)
            hi = jnp.clip(ends_vec[g] - base, 0, _RPW)

            def _row(r, carry):
                return tuple(c + chunk_v[r, pl.ds(cb * _L, _L)]
                             for cb, c in enumerate(carry))
            sums = lax.fori_loop(
                lo, hi, _row,
                tuple(jnp.zeros((_L,), jnp.float32) for _ in range(_CB)))
            for cb in range(_CB):
                acc_v[acc_base + g, pl.ds(cb * _L, _L)] = sums[cb]

    _accumulate(im_hbm, 0, 1, 0)
    _accumulate(s_hbm, 2, 3, _N)

    idx_v[pl.ds(0, _L)] = lane
    idx_v[pl.ds(_L, _L)] = lane + _L
    pltpu.sync_copy(acc_v, shared_acc.at[idx_v], add=True)
    plsc.subcore_barrier()

    pltpu.sync_copy(shared_acc.at[sid], srow_v)
    pltpu.sync_copy(shared_acc.at[pl.ds(_N, _N)], sall_v)
    row = jnp.zeros((_L,), jnp.float32)
    for j in range(_N):
        p = jnp.zeros((_L,), jnp.float32)
        for cb in range(_CB):
            p = p + (srow_v[pl.ds(cb * _L, _L)]
                     * sall_v[j, pl.ds(cb * _L, _L)])
        row = row + jnp.where(lane == j, _allreduce_sum(p, lane), 0.0)
    nclip_f = sizes_v[0, :]
    ncap_f = sizes_v[1, :]
    nclip_i = _allreduce_sum(jnp.where(lane == sid, nclip_f, 0.0), lane)
    counts = nclip_i * ncap_f
    row_v[...] = row / counts
    pltpu.sync_copy(row_v, shared_blk.at[sid])
    plsc.subcore_barrier()

    @pl.when(sid == 0)
    def _():
        pltpu.sync_copy(shared_blk, blk_v)
        d_vec = jnp.zeros((_L,), jnp.float32)
        for i in range(_N):
            d_vec = d_vec + jnp.where(lane == i, blk_v[i, :], 0.0)
        total = jnp.zeros((_L,), jnp.float32)
        for i in range(_N):
            r_i = blk_v[i, :]
            d_i = _allreduce_sum(jnp.where(lane == i, r_i, 0.0), lane)
            off = lane != i
            cost_s = jnp.where(off, jnp.maximum(r_i - d_i, 0.0), 0.0)
            cost_im = jnp.where(off, jnp.maximum(r_i - d_vec, 0.0), 0.0)
            total = total + cost_s + cost_im
        row_v[...] = _allreduce_sum(total, lane)
        pltpu.sync_copy(row_v, out_hbm)


def kernel(im, s, num_clips, num_caps):
    cum_r = jnp.cumsum(num_clips)
    cum_c = jnp.cumsum(num_caps)
    bounds = jnp.stack([cum_r - num_clips, cum_r,
                        cum_c - num_caps, cum_c]).astype(jnp.int32)
    sizes = jnp.stack([num_clips, num_caps]).astype(jnp.float32)

    mesh = plsc.VectorSubcoreMesh(core_axis_name="c", subcore_axis_name="s",
                                  num_cores=1)
    run = functools.partial(
        pl.kernel, mesh=mesh,
        out_type=jax.ShapeDtypeStruct((_L,), jnp.float32),
        scratch_types=[
            pltpu.VMEM((4, _N), jnp.int32),
            pltpu.VMEM((2, _N), jnp.float32),
            pltpu.VMEM((_RPW, _D), jnp.float32),
            pltpu.VMEM((2 * _N, _D), jnp.float32),
            pltpu.VMEM((2 * _N,), jnp.int32),
            pltpu.VMEM((_D,), jnp.float32),
            pltpu.VMEM((_N, _D), jnp.float32),
            pltpu.VMEM((_L,), jnp.float32),
            pltpu.VMEM((_N, _N), jnp.float32),
            pltpu.VMEM_SHARED((2 * _N, _D), jnp.float32),
            pltpu.VMEM_SHARED((_N, _N), jnp.float32),
        ],
    )(_sc_loss_kernel)
    out = run(im, s, bounds, sizes)
    return out[0]
